# Initial kernel scaffold; baseline (speedup 1.0000x reference)
import jax
import jax.numpy as jnp
from jax.experimental import pallas as pl


def kernel(x, edge_index, edge_attr, batch, We1, be1, W11, b11, W12, b12, g1, bb1, We2, be2, W21, b21, W22, b22, g2, bb2, We3, be3, W31, b31, W32, b32, g3, bb3, Wl, bl):
    raise NotImplementedError("write your pallas kernel here")


# speedup vs baseline: 2.5711x; 2.5711x over previous
import functools

import jax
import jax.numpy as jnp
from jax import lax
from jax.experimental import pallas as pl
from jax.experimental.pallas import tpu as pltpu
from jax.experimental.pallas import tpu_sc as plsc

N = 10000
E = 320000
D = 128
H = 128
ED = 16
G = 64

NC = 2
NS = 16
NW = NC * NS
EPW = E // NW
CH = 80
NCHUNK = EPW // CH
NP = 10240
ROWS_PT = NP // NS
ZROWS = 16


def _edge_body(h_hbm, e_hbm, src_hbm, dst_hbm, out_hbm,
               sidx, didx, erows, hrows, zbuf, agg, sem):
    c = lax.axis_index("c")
    s = lax.axis_index("s")
    wid = s * NC + c
    base = wid * EPW

    zvec = jnp.zeros((16,), jnp.float32)

    def zrow(r, _):
        for cc in range(H // 16):
            zbuf[r, pl.ds(cc * 16, 16)] = zvec
        return 0

    lax.fori_loop(0, ZROWS, zrow, 0)

    def zblk(k, _):
        zoff = pl.multiple_of(s * ROWS_PT + k * ZROWS, 8)
        pltpu.sync_copy(zbuf, agg.at[pl.ds(zoff, ZROWS)])
        return 0

    lax.fori_loop(0, ROWS_PT // ZROWS, zblk, 0)
    plsc.subcore_barrier()

    def chunk(i, _):
        off = pl.multiple_of(base + i * CH, 8)
        pltpu.sync_copy(src_hbm.at[pl.ds(off, CH)], sidx)
        pltpu.sync_copy(dst_hbm.at[pl.ds(off, CH)], didx)
        pltpu.sync_copy(e_hbm.at[pl.ds(off, CH)], erows)
        pltpu.async_copy(h_hbm.at[sidx], hrows, sem).wait()

        def row(r, _):
            for cc in range(H // 16):
                sl = pl.ds(cc * 16, 16)
                v = hrows[r, sl] + erows[r, sl]
                hrows[r, sl] = jnp.maximum(v, 0.0)
            return 0

        lax.fori_loop(0, CH, row, 0)
        pltpu.sync_copy(hrows, agg.at[didx], add=True)
        return 0

    lax.fori_loop(0, NCHUNK, chunk, 0)
    plsc.subcore_barrier()

    ooff = pl.multiple_of(s * ROWS_PT, 8)
    pltpu.sync_copy(agg.at[pl.ds(ooff, ROWS_PT)],
                    out_hbm.at[c, pl.ds(ooff, ROWS_PT)])


_sc_mesh = plsc.VectorSubcoreMesh(core_axis_name="c", subcore_axis_name="s")

_edge_agg = pl.kernel(
    _edge_body,
    out_type=jax.ShapeDtypeStruct((NC, NP, H), jnp.float32),
    mesh=_sc_mesh,
    scratch_types=[
        pltpu.VMEM((CH,), jnp.int32),
        pltpu.VMEM((CH,), jnp.int32),
        pltpu.VMEM((CH, H), jnp.float32),
        pltpu.VMEM((CH, H), jnp.float32),
        pltpu.VMEM((ZROWS, H), jnp.float32),
        pltpu.VMEM_SHARED((NP, H), jnp.float32),
        pltpu.SemaphoreType.DMA,
    ],
)


def _elin_body(ea_ref, w_ref, b_ref, o_ref):
    o_ref[...] = (jnp.dot(ea_ref[...], w_ref[...],
                          preferred_element_type=jnp.float32) + b_ref[...])


_BE = 4000


def _elin(ea, W, b):
    return pl.pallas_call(
        _elin_body,
        grid=(E // _BE,),
        in_specs=[pl.BlockSpec((_BE, ED), lambda i: (i, 0)),
                  pl.BlockSpec((ED, H), lambda i: (0, 0)),
                  pl.BlockSpec((1, H), lambda i: (0, 0))],
        out_specs=pl.BlockSpec((_BE, H), lambda i: (i, 0)),
        out_shape=jax.ShapeDtypeStruct((E, H), jnp.float32),
    )(ea, W, b.reshape(1, H))


def _mlp_bn_body(h_ref, agg_ref, w1_ref, b1_ref, w2_ref, b2_ref,
                 g_ref, bb_ref, o_ref):
    z = h_ref[...] + agg_ref[0, :N, :] + agg_ref[1, :N, :]
    a = jnp.maximum(jnp.dot(z, w1_ref[...],
                            preferred_element_type=jnp.float32) + b1_ref[...],
                    0.0)
    y = jnp.dot(a, w2_ref[...], preferred_element_type=jnp.float32) + b2_ref[...]
    mu = jnp.mean(y, axis=0, keepdims=True)
    var = jnp.mean((y - mu) * (y - mu), axis=0, keepdims=True)
    o_ref[...] = jnp.maximum(
        (y - mu) * lax.rsqrt(var + 1e-5) * g_ref[...] + bb_ref[...], 0.0)


def _mlp_bn(h, agg, W1, b1, W2, b2, g, bb):
    return pl.pallas_call(
        _mlp_bn_body,
        out_shape=jax.ShapeDtypeStruct((N, H), jnp.float32),
    )(h, agg, W1, b1.reshape(1, H), W2, b2.reshape(1, H),
      g.reshape(1, H), bb.reshape(1, H))


def _pool_body(h_ref, batch_ref, wl_ref, bl_ref, o_ref):
    b = batch_ref[...]
    gids = lax.broadcasted_iota(jnp.int32, (G, N), 0)
    onehot = (gids == b).astype(jnp.float32)
    sums = jnp.dot(onehot, h_ref[...], preferred_element_type=jnp.float32)
    cnt = jnp.sum(onehot, axis=1, keepdims=True)
    pooled = sums / jnp.maximum(cnt, 1.0)
    o_ref[...] = (jnp.dot(pooled, wl_ref[...],
                          preferred_element_type=jnp.float32) + bl_ref[...])


def _pool(h, batch, Wl, bl):
    return pl.pallas_call(
        _pool_body,
        out_shape=jax.ShapeDtypeStruct((G, 1), jnp.float32),
    )(h, batch.reshape(1, N), Wl, bl.reshape(1, 1))


def kernel(x, edge_index, edge_attr, batch,
           We1, be1, W11, b11, W12, b12, g1, bb1,
           We2, be2, W21, b21, W22, b22, g2, bb2,
           We3, be3, W31, b31, W32, b32, g3, bb3,
           Wl, bl):
    src = edge_index[0]
    dst = edge_index[1]

    h = x
    layers = ((We1, be1, W11, b11, W12, b12, g1, bb1),
              (We2, be2, W21, b21, W22, b22, g2, bb2),
              (We3, be3, W31, b31, W32, b32, g3, bb3))
    for We, be, W1, b1, W2, b2, g, bb in layers:
        e = _elin(edge_attr, We, be)
        agg = _edge_agg(h, e, src, dst)
        h = _mlp_bn(h, agg, W1, b1, W2, b2, g, bb)
    return _pool(h, batch, Wl, bl)


# speedup vs baseline: 4.4582x; 1.7340x over previous
import functools

import jax
import jax.numpy as jnp
from jax import lax
from jax.experimental import pallas as pl
from jax.experimental.pallas import tpu as pltpu
from jax.experimental.pallas import tpu_sc as plsc

N = 10000
E = 320000
D = 128
H = 128
ED = 16
G = 64

NC = 2
NS = 16
NW = NC * NS
EPW = E // NW
CH = 80
NCHUNK = EPW // CH
NP = 10240
ROWS_PT = NP // NS
ZROWS = 16


NBUF = 2
NLAP = NCHUNK // NBUF
NTAIL = NCHUNK - NLAP * NBUF


def _relu_add(hrows, erows, b):
    def row(r, _):
        for cc in range(H // 16):
            sl = pl.ds(cc * 16, 16)
            v = hrows[b, r, sl] + erows[b, r, sl]
            hrows[b, r, sl] = jnp.maximum(v, 0.0)
        return 0

    lax.fori_loop(0, CH, row, 0)


def _edge_body(h_hbm, e_hbm, src_hbm, dst_hbm, out_hbm,
               sidx, didx, erows, hrows, agg,
               semA, semG, semS):
    c = lax.axis_index("c")
    s = lax.axis_index("s")
    wid = s * NC + c
    base = wid * EPW

    zvec = jnp.zeros((16,), jnp.float32)

    def zrow(r, _):
        for cc in range(H // 16):
            hrows[0, r, pl.ds(cc * 16, 16)] = zvec
        return 0

    lax.fori_loop(0, CH, zrow, 0)

    def zblk(k, _):
        zoff = pl.multiple_of(s * ROWS_PT + k * CH, 8)
        pltpu.sync_copy(hrows.at[0], agg.at[pl.ds(zoff, CH)])
        return 0

    lax.fori_loop(0, ROWS_PT // CH, zblk, 0)
    plsc.subcore_barrier()

    def lap(k, _):
        descA = []
        descG = []
        for b in range(NBUF):
            @pl.when(k > 0)
            def _drain():
                pltpu.make_async_copy(
                    hrows.at[b], agg.at[didx.at[b]], semS.at[b]).wait()
            off = pl.multiple_of(base + (k * NBUF + b) * CH, 8)
            descA.append((
                pltpu.async_copy(src_hbm.at[pl.ds(off, CH)], sidx.at[b],
                                 semA.at[b]),
                pltpu.async_copy(dst_hbm.at[pl.ds(off, CH)], didx.at[b],
                                 semA.at[b]),
                pltpu.async_copy(e_hbm.at[pl.ds(off, CH)], erows.at[b],
                                 semA.at[b]),
            ))
        for b in range(NBUF):
            for d in descA[b]:
                d.wait()
            descG.append(
                pltpu.async_copy(h_hbm.at[sidx.at[b]], hrows.at[b],
                                 semG.at[b]))
        for b in range(NBUF):
            descG[b].wait()
            _relu_add(hrows, erows, b)
            pltpu.async_copy(hrows.at[b], agg.at[didx.at[b]], semS.at[b],
                             add=True)
        return 0

    lax.fori_loop(0, NLAP, lap, 0)
    for b in range(NBUF):
        pltpu.make_async_copy(hrows.at[b], agg.at[didx.at[b]],
                              semS.at[b]).wait()
    for t in range(NTAIL):
        off = pl.multiple_of(base + (NLAP * NBUF + t) * CH, 8)
        pltpu.sync_copy(src_hbm.at[pl.ds(off, CH)], sidx.at[0])
        pltpu.sync_copy(dst_hbm.at[pl.ds(off, CH)], didx.at[0])
        pltpu.sync_copy(e_hbm.at[pl.ds(off, CH)], erows.at[0])
        pltpu.async_copy(h_hbm.at[sidx.at[0]], hrows.at[0], semG.at[0]).wait()
        _relu_add(hrows, erows, 0)
        pltpu.sync_copy(hrows.at[0], agg.at[didx.at[0]], add=True)
    plsc.subcore_barrier()

    ooff = pl.multiple_of(s * ROWS_PT, 8)
    pltpu.sync_copy(agg.at[pl.ds(ooff, ROWS_PT)],
                    out_hbm.at[c, pl.ds(ooff, ROWS_PT)])


_sc_mesh = plsc.VectorSubcoreMesh(core_axis_name="c", subcore_axis_name="s")

_edge_agg = pl.kernel(
    _edge_body,
    out_type=jax.ShapeDtypeStruct((NC, NP, H), jnp.float32),
    mesh=_sc_mesh,
    scratch_types=[
        pltpu.VMEM((NBUF, CH), jnp.int32),
        pltpu.VMEM((NBUF, CH), jnp.int32),
        pltpu.VMEM((NBUF, CH, H), jnp.float32),
        pltpu.VMEM((NBUF, CH, H), jnp.float32),
        pltpu.VMEM_SHARED((NP, H), jnp.float32),
        pltpu.SemaphoreType.DMA((NBUF,)),
        pltpu.SemaphoreType.DMA((NBUF,)),
        pltpu.SemaphoreType.DMA((NBUF,)),
    ],
)


def _elin_body(ea_ref, w_ref, b_ref, o_ref):
    o_ref[...] = (jnp.dot(ea_ref[...], w_ref[...],
                          preferred_element_type=jnp.float32) + b_ref[...])


_BE = 4000


def _elin(ea, W, b):
    return pl.pallas_call(
        _elin_body,
        grid=(E // _BE,),
        in_specs=[pl.BlockSpec((_BE, ED), lambda i: (i, 0)),
                  pl.BlockSpec((ED, H), lambda i: (0, 0)),
                  pl.BlockSpec((1, H), lambda i: (0, 0))],
        out_specs=pl.BlockSpec((_BE, H), lambda i: (i, 0)),
        out_shape=jax.ShapeDtypeStruct((E, H), jnp.float32),
    )(ea, W, b.reshape(1, H))


def _mlp_bn_body(h_ref, agg_ref, w1_ref, b1_ref, w2_ref, b2_ref,
                 g_ref, bb_ref, o_ref):
    z = h_ref[...] + agg_ref[0, :N, :] + agg_ref[1, :N, :]
    a = jnp.maximum(jnp.dot(z, w1_ref[...],
                            preferred_element_type=jnp.float32) + b1_ref[...],
                    0.0)
    y = jnp.dot(a, w2_ref[...], preferred_element_type=jnp.float32) + b2_ref[...]
    mu = jnp.mean(y, axis=0, keepdims=True)
    var = jnp.mean((y - mu) * (y - mu), axis=0, keepdims=True)
    o_ref[...] = jnp.maximum(
        (y - mu) * lax.rsqrt(var + 1e-5) * g_ref[...] + bb_ref[...], 0.0)


def _mlp_bn(h, agg, W1, b1, W2, b2, g, bb):
    return pl.pallas_call(
        _mlp_bn_body,
        out_shape=jax.ShapeDtypeStruct((N, H), jnp.float32),
    )(h, agg, W1, b1.reshape(1, H), W2, b2.reshape(1, H),
      g.reshape(1, H), bb.reshape(1, H))


def _pool_body(h_ref, batch_ref, wl_ref, bl_ref, o_ref):
    b = batch_ref[...]
    gids = lax.broadcasted_iota(jnp.int32, (G, N), 0)
    onehot = (gids == b).astype(jnp.float32)
    sums = jnp.dot(onehot, h_ref[...], preferred_element_type=jnp.float32)
    cnt = jnp.sum(onehot, axis=1, keepdims=True)
    pooled = sums / jnp.maximum(cnt, 1.0)
    o_ref[...] = (jnp.dot(pooled, wl_ref[...],
                          preferred_element_type=jnp.float32) + bl_ref[...])


def _pool(h, batch, Wl, bl):
    return pl.pallas_call(
        _pool_body,
        out_shape=jax.ShapeDtypeStruct((G, 1), jnp.float32),
    )(h, batch.reshape(1, N), Wl, bl.reshape(1, 1))


def kernel(x, edge_index, edge_attr, batch,
           We1, be1, W11, b11, W12, b12, g1, bb1,
           We2, be2, W21, b21, W22, b22, g2, bb2,
           We3, be3, W31, b31, W32, b32, g3, bb3,
           Wl, bl):
    src = edge_index[0]
    dst = edge_index[1]

    h = x
    layers = ((We1, be1, W11, b11, W12, b12, g1, bb1),
              (We2, be2, W21, b21, W22, b22, g2, bb2),
              (We3, be3, W31, b31, W32, b32, g3, bb3))
    for We, be, W1, b1, W2, b2, g, bb in layers:
        e = _elin(edge_attr, We, be)
        agg = _edge_agg(h, e, src, dst)
        h = _mlp_bn(h, agg, W1, b1, W2, b2, g, bb)
    return _pool(h, batch, Wl, bl)
